# baseline (speedup 1.0000x reference)
import functools

import jax
import jax.numpy as jnp
from jax import lax
from jax.experimental import pallas as pl
from jax.experimental.pallas import tpu as pltpu
from jax.experimental.pallas import tpu_sc as plsc

_NC = 2
_NS = 16
_NW = _NC * _NS
_L = 16

_TOTAL = 1024 * 512 * 128
_PER_W = _TOTAL // _NW
_CHUNK = 16384
_NCH = _PER_W // _CHUNK
_NBUF = 2
_NG = _NCH // _NBUF


def _sc_body(x_hbm, t_hbm, o_hbm,
             xb0, xb1, tb0, tb1, ob0, ob1,
             sx0, sx1, st0, st1, so0, so1):
    cid = lax.axis_index("c")
    sid = lax.axis_index("s")
    wid = sid * _NC + cid
    base = wid * _PER_W

    xbs = (xb0, xb1)
    tbs = (tb0, tb1)
    obs = (ob0, ob1)
    sxs = (sx0, sx1)
    sts = (st0, st1)
    sos = (so0, so1)

    def in_copies(c, b):
        off = base + c * _CHUNK
        return (
            pltpu.make_async_copy(x_hbm.at[pl.ds(off, _CHUNK)], xbs[b], sxs[b]),
            pltpu.make_async_copy(t_hbm.at[pl.ds(off, _CHUNK)], tbs[b], sts[b]),
        )

    def out_copy(c, b):
        off = base + c * _CHUNK
        return pltpu.make_async_copy(obs[b], o_hbm.at[pl.ds(off, _CHUNK)], sos[b])

    def add_chunk(xb, tb, ob):
        def it(i, carry):
            s = pl.ds(i * _L, _L)
            ob[s] = xb[s] + tb[s]
            return carry
        lax.fori_loop(0, _CHUNK // _L, it, 0, unroll=8)

    for b in range(_NBUF):
        for cp in in_copies(b, b):
            cp.start()

    def group(g, carry):
        for b in range(_NBUF):
            c = g * _NBUF + b
            for cp in in_copies(c, b):
                cp.wait()

            @pl.when(g > 0)
            def _():
                out_copy(c - _NBUF, b).wait()

            add_chunk(xbs[b], tbs[b], obs[b])
            out_copy(c, b).start()

            @pl.when(g < _NG - 1)
            def _():
                for cp in in_copies(c + _NBUF, b):
                    cp.start()

        return carry

    lax.fori_loop(0, _NG, group, 0)

    for b in range(_NBUF):
        out_copy(_NCH - _NBUF + b, b).wait()


_sc_add = functools.partial(
    pl.kernel,
    out_type=jax.ShapeDtypeStruct((_TOTAL,), jnp.float32),
    mesh=plsc.VectorSubcoreMesh(core_axis_name="c", subcore_axis_name="s"),
    scratch_types=(
        [pltpu.VMEM((_CHUNK,), jnp.float32) for _ in range(3 * _NBUF)]
        + [pltpu.SemaphoreType.DMA for _ in range(3 * _NBUF)]
    ),
)(_sc_body)


def kernel(x, table):
    B, N, D = x.shape
    out = _sc_add(x.reshape(_TOTAL), table.reshape(_TOTAL))
    return out.reshape(B, N, D)


# speedup vs baseline: 1.2633x; 1.2633x over previous
import functools

import jax
import jax.numpy as jnp
from jax import lax
from jax.experimental import pallas as pl
from jax.experimental.pallas import tpu as pltpu
from jax.experimental.pallas import tpu_sc as plsc

_NC = 2
_NS = 16
_NW = _NC * _NS
_L = 16

_TOTAL = 1024 * 512 * 128
_PER_W = _TOTAL // _NW
_CHUNK = 16384
_NCH = _PER_W // _CHUNK
_NBUF = 2
_NG = _NCH // _NBUF


_ROW = 512 * 128
_CPR = _ROW // _CHUNK
_RPW = 1024 // _NW


def _sc_body(x_hbm, t_hbm, o_hbm,
             xb0, xb1, tb0, tb1, ob0, ob1,
             sx0, sx1, st0, st1, so0, so1):
    cid = lax.axis_index("c")
    sid = lax.axis_index("s")
    wid = sid * _NC + cid
    base = wid * _PER_W
    row0 = wid * _RPW

    xbs = (xb0, xb1)
    tbs = (tb0, tb1)
    obs = (ob0, ob1)
    sxs = (sx0, sx1)
    sts = (st0, st1)
    sos = (so0, so1)

    def in_copies(c, b):
        off = base + c * _CHUNK
        row = row0 + c // _CPR
        k0 = (c % _CPR) * _CHUNK
        return (
            pltpu.make_async_copy(x_hbm.at[pl.ds(off, _CHUNK)], xbs[b], sxs[b]),
            pltpu.make_async_copy(t_hbm.at[row, pl.ds(k0, _CHUNK)], tbs[b], sts[b]),
        )

    def out_copy(c, b):
        off = base + c * _CHUNK
        return pltpu.make_async_copy(obs[b], o_hbm.at[pl.ds(off, _CHUNK)], sos[b])

    def add_chunk(xb, tb, ob):
        def it(i, carry):
            s = pl.ds(i * _L, _L)
            ob[s] = xb[s] + tb[s]
            return carry
        lax.fori_loop(0, _CHUNK // _L, it, 0, unroll=8)

    for b in range(_NBUF):
        for cp in in_copies(b, b):
            cp.start()

    def group(g, carry):
        for b in range(_NBUF):
            c = g * _NBUF + b
            for cp in in_copies(c, b):
                cp.wait()

            @pl.when(g > 0)
            def _():
                out_copy(c - _NBUF, b).wait()

            add_chunk(xbs[b], tbs[b], obs[b])
            out_copy(c, b).start()

            @pl.when(g < _NG - 1)
            def _():
                for cp in in_copies(c + _NBUF, b):
                    cp.start()

        return carry

    lax.fori_loop(0, _NG, group, 0)

    for b in range(_NBUF):
        out_copy(_NCH - _NBUF + b, b).wait()


_sc_add = functools.partial(
    pl.kernel,
    out_type=jax.ShapeDtypeStruct((_TOTAL,), jnp.float32),
    mesh=plsc.VectorSubcoreMesh(core_axis_name="c", subcore_axis_name="s"),
    scratch_types=(
        [pltpu.VMEM((_CHUNK,), jnp.float32) for _ in range(3 * _NBUF)]
        + [pltpu.SemaphoreType.DMA for _ in range(3 * _NBUF)]
    ),
)(_sc_body)


def kernel(x, table):
    B, N, D = x.shape
    out = _sc_add(x.reshape(_TOTAL), table)
    return out.reshape(B, N, D)
